# baseline (speedup 1.0000x reference)
import functools

import jax
import jax.numpy as jnp
from jax import lax
from jax.experimental import pallas as pl
from jax.experimental.pallas import tpu as pltpu
from jax.experimental.pallas import tpu_sc as plsc

N, E, D, DE = 10000, 320000, 128, 16
CUN = 320
NC, NS = 2, 16
NW = NC * NS
L = 16
CH = 128
ZCH = 80
ZCHUNKS = N // ZCH
CUCHUNKS = CUN // ZCH
NSLICE = 4
ESL = E // NSLICE


def _vector_mesh():
    return plsc.VectorSubcoreMesh(core_axis_name="c", subcore_axis_name="s")


_SC_PARAMS = pltpu.CompilerParams(needs_layout_passes=False)
_SC_PARAMS_UNTILED = pltpu.CompilerParams(needs_layout_passes=False,
                                          use_tc_tiling_on_sc=False)


@jax.jit
def _sc_gather(table, idx2, px, py, pz, row, col):
    ne = row.shape[0]
    gchunks_all = (2 * ne) // CH
    gchunks = -(-gchunks_all // NW)
    echunks_all = ne // CH
    echunks = -(-echunks_all // NW)

    @functools.partial(
        pl.kernel,
        mesh=_vector_mesh(),
        compiler_params=_SC_PARAMS_UNTILED,
        out_type=[
            jax.ShapeDtypeStruct((2 * ne, D), jnp.bfloat16),
            jax.ShapeDtypeStruct((8, ne), jnp.float32),
        ],
        scratch_types=[
            pltpu.VMEM((CH,), jnp.int32),
            pltpu.VMEM((CH, D), jnp.bfloat16),
            pltpu.VMEM((N,), jnp.float32),
            pltpu.VMEM((N,), jnp.float32),
            pltpu.VMEM((N,), jnp.float32),
            pltpu.VMEM((CH,), jnp.int32),
            pltpu.VMEM((CH,), jnp.int32),
            pltpu.VMEM((8, CH), jnp.float32),
            pltpu.SemaphoreType.DMA,
        ],
    )
    def gk(table_hbm, idx_hbm, px_hbm, py_hbm, pz_hbm, row_hbm, col_hbm,
           out_hbm, cdr_hbm,
           idx_v, rows_v, px_v, py_v, pz_v, r_v, c_v, geo_v, sem):
        wid = lax.axis_index("c") * NS + lax.axis_index("s")

        pltpu.sync_copy(px_hbm, px_v)
        pltpu.sync_copy(py_hbm, py_v)
        pltpu.sync_copy(pz_hbm, pz_v)

        zero16 = jnp.zeros((L,), jnp.float32)

        @pl.loop(5, 8)
        def _(r):
            @pl.loop(0, CH // L)
            def _(cc):
                geo_v[r, pl.ds(cc * L, L)] = zero16

        @pl.loop(0, echunks)
        def _(ch):
            cid = wid + ch * NW

            @pl.when(cid < echunks_all)
            def _():
                off = cid * CH
                pltpu.sync_copy(row_hbm.at[pl.ds(off, CH)], r_v)
                pltpu.sync_copy(col_hbm.at[pl.ds(off, CH)], c_v)

                @pl.loop(0, CH // L)
                def _(k):
                    sl = pl.ds(k * L, L)
                    ir = r_v[sl]
                    ic = c_v[sl]
                    dx = (plsc.load_gather(px_v, [ir])
                          - plsc.load_gather(px_v, [ic]))
                    dy = (plsc.load_gather(py_v, [ir])
                          - plsc.load_gather(py_v, [ic]))
                    dz = (plsc.load_gather(pz_v, [ir])
                          - plsc.load_gather(pz_v, [ic]))
                    geo_v[0, sl] = dx
                    geo_v[1, sl] = dy
                    geo_v[2, sl] = dz
                    geo_v[3, sl] = dx * dx + dy * dy + dz * dz
                    geo_v[4, sl] = lax.convert_element_type(
                        lax.bitwise_and(ir, 31), jnp.float32)

                pltpu.sync_copy(geo_v, cdr_hbm.at[:, pl.ds(off, CH)])

        @pl.loop(0, gchunks)
        def _(ch):
            cid = wid + ch * NW

            @pl.when(cid < gchunks_all)
            def _():
                off = cid * CH
                pltpu.sync_copy(idx_hbm.at[pl.ds(off, CH)], idx_v)
                pltpu.async_copy(table_hbm.at[idx_v], rows_v, sem).wait()
                pltpu.sync_copy(rows_v, out_hbm.at[pl.ds(off, CH)])

    return gk(table, idx2, px, py, pz, row, col)


@jax.jit
def _sc_scatter(mvals, cus, row):
    ne = row.shape[0]
    echunks_all = ne // CH
    echunks = -(-echunks_all // NW)

    @functools.partial(
        pl.kernel,
        mesh=_vector_mesh(),
        compiler_params=_SC_PARAMS,
        out_type=[
            jax.ShapeDtypeStruct((NC, N, D), jnp.float32),
            jax.ShapeDtypeStruct((NC, CUN, D), jnp.float32),
        ],
        scratch_types=[
            pltpu.VMEM((CH,), jnp.int32),
            pltpu.VMEM((CH,), jnp.int32),
            pltpu.VMEM((CH, D), jnp.float32),
            pltpu.VMEM((CH, D), jnp.float32),
            pltpu.VMEM((8, CH), jnp.float32),
            pltpu.VMEM_SHARED((N, D), jnp.float32),
            pltpu.VMEM_SHARED((CUN, D), jnp.float32),
            pltpu.SemaphoreType.DMA,
        ],
    )
    def sk(mvals_hbm, cus_hbm, idx_hbm, outm_hbm, outcu_hbm,
           idx_v, cuidx_v, mv, cuv, cus_v, macc, cuacc, sem):
        c = lax.axis_index("c")
        s = lax.axis_index("s")
        wid = c * NS + s

        zero16 = jnp.zeros((L,), jnp.float32)
        one16 = jnp.ones((L,), jnp.float32)

        @pl.loop(0, CH)
        def _(r):
            @pl.loop(0, D // L)
            def _(cc):
                mv[r, pl.ds(cc * L, L)] = zero16
                cuv[r, pl.ds(cc * L, L)] = zero16

        @pl.loop(0, ZCHUNKS)
        def _(z):
            @pl.when(lax.rem(z, NS) == s)
            def _():
                pltpu.sync_copy(mv.at[pl.ds(0, ZCH)],
                                macc.at[pl.ds(z * ZCH, ZCH)])

        @pl.loop(0, CUCHUNKS)
        def _(z):
            @pl.when(z == s)
            def _():
                pltpu.sync_copy(mv.at[pl.ds(0, ZCH)],
                                cuacc.at[pl.ds(z * ZCH, ZCH)])

        plsc.subcore_barrier()

        lanes16 = lax.iota(jnp.int32, L)

        @pl.loop(0, echunks)
        def _(ch):
            cid = wid + ch * NW

            @pl.when(cid < echunks_all)
            def _():
                off = cid * CH
                pltpu.sync_copy(idx_hbm.at[pl.ds(off, CH)], idx_v)
                pltpu.sync_copy(mvals_hbm.at[pl.ds(off, CH)], mv)
                pltpu.sync_copy(cus_hbm.at[:, pl.ds(off, CH)], cus_v)

                @pl.loop(0, CH // L)
                def _(k):
                    sl = pl.ds(k * L, L)
                    r16 = idx_v[sl]
                    rowi = lanes16 + k * L
                    lane = lax.shift_left(lax.bitwise_and(r16, 31), 2)
                    plsc.store_scatter(cuv, [rowi, lane], cus_v[0, sl])
                    plsc.store_scatter(cuv, [rowi, lane + 1], cus_v[1, sl])
                    plsc.store_scatter(cuv, [rowi, lane + 2], cus_v[2, sl])
                    plsc.store_scatter(cuv, [rowi, lane + 3], one16)
                    cuidx_v[sl] = lax.shift_right_logical(r16, 5)

                pltpu.sync_copy(mv, macc.at[idx_v], add=True)
                pltpu.sync_copy(cuv, cuacc.at[cuidx_v], add=True)

                @pl.loop(0, CH // L)
                def _(k):
                    sl = pl.ds(k * L, L)
                    r16 = idx_v[sl]
                    rowi = lanes16 + k * L
                    lane = lax.shift_left(lax.bitwise_and(r16, 31), 2)
                    plsc.store_scatter(cuv, [rowi, lane], zero16)
                    plsc.store_scatter(cuv, [rowi, lane + 1], zero16)
                    plsc.store_scatter(cuv, [rowi, lane + 2], zero16)
                    plsc.store_scatter(cuv, [rowi, lane + 3], zero16)

        plsc.subcore_barrier()

        @pl.loop(0, ZCHUNKS)
        def _(z):
            @pl.when(lax.rem(z, NS) == s)
            def _():
                pltpu.sync_copy(macc.at[pl.ds(z * ZCH, ZCH)],
                                outm_hbm.at[c, pl.ds(z * ZCH, ZCH)])

        @pl.loop(0, CUCHUNKS)
        def _(z):
            @pl.when(z == s)
            def _():
                pltpu.sync_copy(cuacc.at[pl.ds(z * ZCH, ZCH)],
                                outcu_hbm.at[c, pl.ds(z * ZCH, ZCH)])

    return sk(mvals, cus, row)


BE = 3200


def _silu(x):
    return x * jax.nn.sigmoid(x)


def _bf(x):
    return x.astype(jnp.bfloat16)


def _edge_body(gr_ref, gc_ref, ea_ref, cdr_ref, w1r_ref, w1c_ref, w1a_ref,
               w1rad_ref, b1_ref, w2_ref, b2_ref, wc1_ref, bc1_ref, wc2_ref,
               m_ref, cus_ref):
    eye8 = jnp.eye(8, dtype=jnp.float32)
    cdrT = lax.dot_general(cdr_ref[...], eye8, (((0,), (0,)), ((), ())),
                           preferred_element_type=jnp.float32)
    cd = cdrT[:, 0:3]
    radial = cdrT[:, 3:4]

    t1 = (jnp.dot(gr_ref[...], w1r_ref[...],
                  preferred_element_type=jnp.float32)
          + jnp.dot(gc_ref[...], w1c_ref[...],
                    preferred_element_type=jnp.float32)
          + jnp.dot(_bf(ea_ref[...]), w1a_ref[...],
                    preferred_element_type=jnp.float32)
          + radial * w1rad_ref[...]
          + b1_ref[...])
    x = _silu(t1)
    m = _silu(jnp.dot(_bf(x), w2_ref[...], preferred_element_type=jnp.float32)
              + b2_ref[...])
    c1 = _silu(jnp.dot(_bf(m), wc1_ref[...], preferred_element_type=jnp.float32)
               + bc1_ref[...])
    w = jnp.sum(c1 * wc2_ref[...], axis=1, keepdims=True)
    cu = cd * (w * lax.rsqrt(radial + 1e-8))

    m_ref[...] = m
    cuT = jnp.concatenate([cu, jnp.zeros((BE, 5), jnp.float32)], axis=1)
    cus_ref[...] = lax.dot_general(eye8, cuT, (((1,), (1,)), ((), ())),
                                   preferred_element_type=jnp.float32)


def _edge_mlp(gathered, edge_attr, cdr, w1r, w1c, w1a, w1rad, b1, w2, b2,
              wc1, bc1, wc2):
    ne = edge_attr.shape[0]
    nb = ne // BE
    full = lambda shape: pl.BlockSpec(shape, lambda i: (0, 0))
    return pl.pallas_call(
        _edge_body,
        grid=(nb,),
        in_specs=[
            pl.BlockSpec((BE, D), lambda i: (i, 0)),
            pl.BlockSpec((BE, D), lambda i: (i + nb, 0)),
            pl.BlockSpec((BE, DE), lambda i: (i, 0)),
            pl.BlockSpec((8, BE), lambda i: (0, i)),
            full((D, D)), full((D, D)), full((DE, D)), full((1, D)),
            full((1, D)), full((D, D)), full((1, D)),
            full((D, D)), full((1, D)), full((1, D)),
        ],
        out_specs=[
            pl.BlockSpec((BE, D), lambda i: (i, 0)),
            pl.BlockSpec((8, BE), lambda i: (0, i)),
        ],
        out_shape=[
            jax.ShapeDtypeStruct((ne, D), jnp.float32),
            jax.ShapeDtypeStruct((8, ne), jnp.float32),
        ],
    )(gathered, gathered, edge_attr, cdr, w1r, w1c, w1a, w1rad, b1, w2, b2,
      wc1, bc1, wc2)


BN = 2000


def _node_body(p_ref, q_ref, h_ref, pos_ref, wn1h_ref, wn1m_ref, bn1_ref,
               wn2_ref, bn2_ref, hnew_ref, posnew_ref):
    m_i = p_ref[...]
    q = q_ref[...]
    pu = q[:, 0:3]
    cnt = q[:, 3:4]
    h = h_ref[...]
    t = _silu(jnp.dot(_bf(h), wn1h_ref[...], preferred_element_type=jnp.float32)
              + jnp.dot(_bf(m_i), wn1m_ref[...],
                        preferred_element_type=jnp.float32)
              + bn1_ref[...])
    hnew_ref[...] = h + jnp.dot(_bf(t), wn2_ref[...],
                                preferred_element_type=jnp.float32) + bn2_ref[...]
    posnew_ref[...] = pos_ref[...] + pu / (cnt + 1e-6)


def _node_mlp(p, q, h, pos, wn1h, wn1m, bn1, wn2, bn2):
    nb = N // BN
    full = lambda shape: pl.BlockSpec(shape, lambda i: (0, 0))
    return pl.pallas_call(
        _node_body,
        grid=(nb,),
        in_specs=[
            pl.BlockSpec((BN, D), lambda i: (i, 0)),
            pl.BlockSpec((BN, 4), lambda i: (i, 0)),
            pl.BlockSpec((BN, D), lambda i: (i, 0)),
            pl.BlockSpec((BN, 3), lambda i: (i, 0)),
            full((D, D)), full((D, D)), full((1, D)),
            full((D, D)), full((1, D)),
        ],
        out_specs=[
            pl.BlockSpec((BN, D), lambda i: (i, 0)),
            pl.BlockSpec((BN, 3), lambda i: (i, 0)),
        ],
        out_shape=[
            jax.ShapeDtypeStruct((N, D), jnp.float32),
            jax.ShapeDtypeStruct((N, 3), jnp.float32),
        ],
    )(p, q, h, pos, wn1h, wn1m, bn1, wn2, bn2)


def kernel(h, pos, edge_index, edge_attr, W_e1, b_e1, W_e2, b_e2,
           W_n1, b_n1, W_n2, b_n2, W_c1, b_c1, W_c2):
    row, col = edge_index[0], edge_index[1]
    px, py, pz = pos[:, 0], pos[:, 1], pos[:, 2]
    h_bf = h.astype(jnp.bfloat16)

    bf = lambda w: w.astype(jnp.bfloat16)
    ew = (bf(W_e1[:D]), bf(W_e1[D:2 * D]), bf(W_e1[2 * D + 1:]),
          W_e1[2 * D:2 * D + 1], b_e1.reshape(1, D),
          bf(W_e2), b_e2.reshape(1, D),
          bf(W_c1), b_c1.reshape(1, D), W_c2.reshape(1, D))

    outms, outcus = [], []
    for k in range(NSLICE):
        sl = slice(k * ESL, (k + 1) * ESL)
        row_k, col_k = row[sl], col[sl]
        idx2_k = jnp.concatenate([row_k, col_k])
        gathered, cdr = _sc_gather(h_bf, idx2_k, px, py, pz, row_k, col_k)
        mvals, cus = _edge_mlp(gathered, edge_attr[sl], cdr, *ew)
        outm, outcu = _sc_scatter(mvals, cus, row_k)
        outms.append(outm)
        outcus.append(outcu)

    m_i = sum(o[0] + o[1] for o in outms)
    qacc = sum(o[0] + o[1] for o in outcus)
    q = qacc.reshape(CUN * 32, 4)[:N]
    h_new, pos_new = _node_mlp(m_i, q, h, pos,
                               bf(W_n1[:D]), bf(W_n1[D:]), b_n1.reshape(1, D),
                               bf(W_n2), b_n2.reshape(1, D))
    return (h_new, pos_new)


# speedup vs baseline: 1.7337x; 1.7337x over previous
import functools

import jax
import jax.numpy as jnp
from jax import lax
from jax.experimental import pallas as pl
from jax.experimental.pallas import tpu as pltpu
from jax.experimental.pallas import tpu_sc as plsc

N, E, D, DE = 10000, 320000, 128, 16
CUN = 320
NC, NS = 2, 16
NW = NC * NS
L = 16
CH = 128
ZCH = 80
ZCHUNKS = N // ZCH
CUCHUNKS = CUN // ZCH
NSLICE = 4
ESL = E // NSLICE


def _vector_mesh():
    return plsc.VectorSubcoreMesh(core_axis_name="c", subcore_axis_name="s")


_SC_PARAMS = pltpu.CompilerParams(needs_layout_passes=False)
_SC_PARAMS_UNTILED = pltpu.CompilerParams(needs_layout_passes=False,
                                          use_tc_tiling_on_sc=False)


GCH = 40
NB = 4


@jax.jit
def _sc_gather(table, idx2, px, py, pz, row, col):
    ne = row.shape[0]
    rpt = (2 * ne) // NW
    nch = rpt // GCH
    assert rpt % GCH == 0 and nch > 2 * NB
    echunks_all = ne // CH
    echunks = -(-echunks_all // NW)

    @functools.partial(
        pl.kernel,
        mesh=_vector_mesh(),
        compiler_params=_SC_PARAMS,
        out_type=[
            jax.ShapeDtypeStruct((2 * ne, D), jnp.float32),
            jax.ShapeDtypeStruct((8, ne), jnp.float32),
        ],
        scratch_types=[
            pltpu.VMEM((rpt,), jnp.int32),
        ] + [pltpu.VMEM((GCH, D), jnp.float32) for _ in range(NB)] + [
            pltpu.VMEM((N,), jnp.float32),
            pltpu.VMEM((N,), jnp.float32),
            pltpu.VMEM((N,), jnp.float32),
            pltpu.VMEM((CH,), jnp.int32),
            pltpu.VMEM((CH,), jnp.int32),
            pltpu.VMEM((8, CH), jnp.float32),
        ] + [pltpu.SemaphoreType.DMA for _ in range(2 * NB)],
    )
    def gk(table_hbm, idx_hbm, px_hbm, py_hbm, pz_hbm, row_hbm, col_hbm,
           out_hbm, cdr_hbm,
           idx_v, rb0, rb1, rb2, rb3, px_v, py_v, pz_v, r_v, c_v, geo_v,
           g0, g1, g2, g3, o0, o1, o2, o3):
        rbufs = (rb0, rb1, rb2, rb3)
        gsems = (g0, g1, g2, g3)
        osems = (o0, o1, o2, o3)
        wid = lax.axis_index("c") * NS + lax.axis_index("s")

        pltpu.sync_copy(px_hbm, px_v)
        pltpu.sync_copy(py_hbm, py_v)
        pltpu.sync_copy(pz_hbm, pz_v)

        zero16 = jnp.zeros((L,), jnp.float32)

        @pl.loop(5, 8)
        def _(r):
            @pl.loop(0, CH // L)
            def _(cc):
                geo_v[r, pl.ds(cc * L, L)] = zero16

        @pl.loop(0, echunks)
        def _(ch):
            cid = wid + ch * NW

            @pl.when(cid < echunks_all)
            def _():
                off = cid * CH
                pltpu.sync_copy(row_hbm.at[pl.ds(off, CH)], r_v)
                pltpu.sync_copy(col_hbm.at[pl.ds(off, CH)], c_v)

                @pl.loop(0, CH // L)
                def _(k):
                    sl = pl.ds(k * L, L)
                    ir = r_v[sl]
                    ic = c_v[sl]
                    dx = (plsc.load_gather(px_v, [ir])
                          - plsc.load_gather(px_v, [ic]))
                    dy = (plsc.load_gather(py_v, [ir])
                          - plsc.load_gather(py_v, [ic]))
                    dz = (plsc.load_gather(pz_v, [ir])
                          - plsc.load_gather(pz_v, [ic]))
                    geo_v[0, sl] = dx
                    geo_v[1, sl] = dy
                    geo_v[2, sl] = dz
                    geo_v[3, sl] = dx * dx + dy * dy + dz * dz
                    geo_v[4, sl] = lax.convert_element_type(
                        lax.bitwise_and(ir, 31), jnp.float32)

                pltpu.sync_copy(geo_v, cdr_hbm.at[:, pl.ds(off, CH)])

        gbase = wid * rpt
        pltpu.sync_copy(idx_hbm.at[pl.ds(gbase, rpt)], idx_v)

        def start_gather(cid, b):
            pltpu.async_copy(table_hbm.at[idx_v.at[pl.ds(cid * GCH, GCH)]],
                             rbufs[b], gsems[b])

        def start_out(cid, b):
            pltpu.async_copy(rbufs[b],
                             out_hbm.at[pl.ds(gbase + cid * GCH, GCH)],
                             osems[b])

        for b in range(NB):
            start_gather(b, b)

        @pl.loop(0, -(-nch // NB))
        def _(t):
            for b in range(NB):
                cid = t * NB + b

                @pl.when(cid < nch)
                def _():
                    pltpu.make_async_copy(table_hbm.at[pl.ds(0, GCH)],
                                          rbufs[b], gsems[b]).wait()
                    start_out(cid, b)

                @pl.when(cid + NB < nch)
                def _():
                    pltpu.make_async_copy(rbufs[b],
                                          out_hbm.at[pl.ds(0, GCH)],
                                          osems[b]).wait()
                    start_gather(cid + NB, b)

        for b in range(NB):
            pltpu.make_async_copy(rbufs[b], out_hbm.at[pl.ds(0, GCH)],
                                  osems[b]).wait()

    return gk(table, idx2, px, py, pz, row, col)


@jax.jit
def _sc_scatter(mvals, cus, row):
    ne = row.shape[0]
    echunks_all = ne // CH
    echunks = -(-echunks_all // NW)

    @functools.partial(
        pl.kernel,
        mesh=_vector_mesh(),
        compiler_params=_SC_PARAMS,
        out_type=[
            jax.ShapeDtypeStruct((NC, N, D), jnp.float32),
            jax.ShapeDtypeStruct((NC, CUN, D), jnp.float32),
        ],
        scratch_types=[
            pltpu.VMEM((CH,), jnp.int32),
            pltpu.VMEM((CH,), jnp.int32),
            pltpu.VMEM((CH, D), jnp.float32),
            pltpu.VMEM((CH, D), jnp.float32),
            pltpu.VMEM((8, CH), jnp.float32),
            pltpu.VMEM_SHARED((N, D), jnp.float32),
            pltpu.VMEM_SHARED((CUN, D), jnp.float32),
            pltpu.SemaphoreType.DMA,
        ],
    )
    def sk(mvals_hbm, cus_hbm, idx_hbm, outm_hbm, outcu_hbm,
           idx_v, cuidx_v, mv, cuv, cus_v, macc, cuacc, sem):
        c = lax.axis_index("c")
        s = lax.axis_index("s")
        wid = c * NS + s

        zero16 = jnp.zeros((L,), jnp.float32)
        one16 = jnp.ones((L,), jnp.float32)

        @pl.loop(0, CH)
        def _(r):
            @pl.loop(0, D // L)
            def _(cc):
                mv[r, pl.ds(cc * L, L)] = zero16
                cuv[r, pl.ds(cc * L, L)] = zero16

        @pl.loop(0, ZCHUNKS)
        def _(z):
            @pl.when(lax.rem(z, NS) == s)
            def _():
                pltpu.sync_copy(mv.at[pl.ds(0, ZCH)],
                                macc.at[pl.ds(z * ZCH, ZCH)])

        @pl.loop(0, CUCHUNKS)
        def _(z):
            @pl.when(z == s)
            def _():
                pltpu.sync_copy(mv.at[pl.ds(0, ZCH)],
                                cuacc.at[pl.ds(z * ZCH, ZCH)])

        plsc.subcore_barrier()

        lanes16 = lax.iota(jnp.int32, L)

        @pl.loop(0, echunks)
        def _(ch):
            cid = wid + ch * NW

            @pl.when(cid < echunks_all)
            def _():
                off = cid * CH
                pltpu.sync_copy(idx_hbm.at[pl.ds(off, CH)], idx_v)
                pltpu.sync_copy(mvals_hbm.at[pl.ds(off, CH)], mv)
                pltpu.sync_copy(cus_hbm.at[:, pl.ds(off, CH)], cus_v)

                @pl.loop(0, CH // L)
                def _(k):
                    sl = pl.ds(k * L, L)
                    r16 = idx_v[sl]
                    rowi = lanes16 + k * L
                    lane = lax.shift_left(lax.bitwise_and(r16, 31), 2)
                    plsc.store_scatter(cuv, [rowi, lane], cus_v[0, sl])
                    plsc.store_scatter(cuv, [rowi, lane + 1], cus_v[1, sl])
                    plsc.store_scatter(cuv, [rowi, lane + 2], cus_v[2, sl])
                    plsc.store_scatter(cuv, [rowi, lane + 3], one16)
                    cuidx_v[sl] = lax.shift_right_logical(r16, 5)

                pltpu.sync_copy(mv, macc.at[idx_v], add=True)
                pltpu.sync_copy(cuv, cuacc.at[cuidx_v], add=True)

                @pl.loop(0, CH // L)
                def _(k):
                    sl = pl.ds(k * L, L)
                    r16 = idx_v[sl]
                    rowi = lanes16 + k * L
                    lane = lax.shift_left(lax.bitwise_and(r16, 31), 2)
                    plsc.store_scatter(cuv, [rowi, lane], zero16)
                    plsc.store_scatter(cuv, [rowi, lane + 1], zero16)
                    plsc.store_scatter(cuv, [rowi, lane + 2], zero16)
                    plsc.store_scatter(cuv, [rowi, lane + 3], zero16)

        plsc.subcore_barrier()

        @pl.loop(0, ZCHUNKS)
        def _(z):
            @pl.when(lax.rem(z, NS) == s)
            def _():
                pltpu.sync_copy(macc.at[pl.ds(z * ZCH, ZCH)],
                                outm_hbm.at[c, pl.ds(z * ZCH, ZCH)])

        @pl.loop(0, CUCHUNKS)
        def _(z):
            @pl.when(z == s)
            def _():
                pltpu.sync_copy(cuacc.at[pl.ds(z * ZCH, ZCH)],
                                outcu_hbm.at[c, pl.ds(z * ZCH, ZCH)])

    return sk(mvals, cus, row)


BE = 3200


def _silu(x):
    return x * jax.nn.sigmoid(x)


def _bf(x):
    return x.astype(jnp.bfloat16)


def _edge_body(gr_ref, gc_ref, ea_ref, cdr_ref, w1r_ref, w1c_ref, w1a_ref,
               w1rad_ref, b1_ref, w2_ref, b2_ref, wc1_ref, bc1_ref, wc2_ref,
               m_ref, cus_ref):
    eye8 = jnp.eye(8, dtype=jnp.float32)
    cdrT = lax.dot_general(cdr_ref[...], eye8, (((0,), (0,)), ((), ())),
                           preferred_element_type=jnp.float32)
    cd = cdrT[:, 0:3]
    radial = cdrT[:, 3:4]

    t1 = (jnp.dot(_bf(gr_ref[...]), w1r_ref[...],
                  preferred_element_type=jnp.float32)
          + jnp.dot(_bf(gc_ref[...]), w1c_ref[...],
                    preferred_element_type=jnp.float32)
          + jnp.dot(_bf(ea_ref[...]), w1a_ref[...],
                    preferred_element_type=jnp.float32)
          + radial * w1rad_ref[...]
          + b1_ref[...])
    x = _silu(t1)
    m = _silu(jnp.dot(_bf(x), w2_ref[...], preferred_element_type=jnp.float32)
              + b2_ref[...])
    c1 = _silu(jnp.dot(_bf(m), wc1_ref[...], preferred_element_type=jnp.float32)
               + bc1_ref[...])
    w = jnp.sum(c1 * wc2_ref[...], axis=1, keepdims=True)
    cu = cd * (w * lax.rsqrt(radial + 1e-8))

    m_ref[...] = m
    cuT = jnp.concatenate([cu, jnp.zeros((BE, 5), jnp.float32)], axis=1)
    cus_ref[...] = lax.dot_general(eye8, cuT, (((1,), (1,)), ((), ())),
                                   preferred_element_type=jnp.float32)


def _edge_mlp(gathered, edge_attr, cdr, w1r, w1c, w1a, w1rad, b1, w2, b2,
              wc1, bc1, wc2):
    ne = edge_attr.shape[0]
    nb = ne // BE
    full = lambda shape: pl.BlockSpec(shape, lambda i: (0, 0))
    return pl.pallas_call(
        _edge_body,
        grid=(nb,),
        in_specs=[
            pl.BlockSpec((BE, D), lambda i: (i, 0)),
            pl.BlockSpec((BE, D), lambda i: (i + nb, 0)),
            pl.BlockSpec((BE, DE), lambda i: (i, 0)),
            pl.BlockSpec((8, BE), lambda i: (0, i)),
            full((D, D)), full((D, D)), full((DE, D)), full((1, D)),
            full((1, D)), full((D, D)), full((1, D)),
            full((D, D)), full((1, D)), full((1, D)),
        ],
        out_specs=[
            pl.BlockSpec((BE, D), lambda i: (i, 0)),
            pl.BlockSpec((8, BE), lambda i: (0, i)),
        ],
        out_shape=[
            jax.ShapeDtypeStruct((ne, D), jnp.float32),
            jax.ShapeDtypeStruct((8, ne), jnp.float32),
        ],
    )(gathered, gathered, edge_attr, cdr, w1r, w1c, w1a, w1rad, b1, w2, b2,
      wc1, bc1, wc2)


BN = 2000


def _node_body(p_ref, q_ref, h_ref, pos_ref, wn1h_ref, wn1m_ref, bn1_ref,
               wn2_ref, bn2_ref, hnew_ref, posnew_ref):
    m_i = p_ref[...]
    q = q_ref[...]
    pu = q[:, 0:3]
    cnt = q[:, 3:4]
    h = h_ref[...]
    t = _silu(jnp.dot(_bf(h), wn1h_ref[...], preferred_element_type=jnp.float32)
              + jnp.dot(_bf(m_i), wn1m_ref[...],
                        preferred_element_type=jnp.float32)
              + bn1_ref[...])
    hnew_ref[...] = h + jnp.dot(_bf(t), wn2_ref[...],
                                preferred_element_type=jnp.float32) + bn2_ref[...]
    posnew_ref[...] = pos_ref[...] + pu / (cnt + 1e-6)


def _node_mlp(p, q, h, pos, wn1h, wn1m, bn1, wn2, bn2):
    nb = N // BN
    full = lambda shape: pl.BlockSpec(shape, lambda i: (0, 0))
    return pl.pallas_call(
        _node_body,
        grid=(nb,),
        in_specs=[
            pl.BlockSpec((BN, D), lambda i: (i, 0)),
            pl.BlockSpec((BN, 4), lambda i: (i, 0)),
            pl.BlockSpec((BN, D), lambda i: (i, 0)),
            pl.BlockSpec((BN, 3), lambda i: (i, 0)),
            full((D, D)), full((D, D)), full((1, D)),
            full((D, D)), full((1, D)),
        ],
        out_specs=[
            pl.BlockSpec((BN, D), lambda i: (i, 0)),
            pl.BlockSpec((BN, 3), lambda i: (i, 0)),
        ],
        out_shape=[
            jax.ShapeDtypeStruct((N, D), jnp.float32),
            jax.ShapeDtypeStruct((N, 3), jnp.float32),
        ],
    )(p, q, h, pos, wn1h, wn1m, bn1, wn2, bn2)


def kernel(h, pos, edge_index, edge_attr, W_e1, b_e1, W_e2, b_e2,
           W_n1, b_n1, W_n2, b_n2, W_c1, b_c1, W_c2):
    row, col = edge_index[0], edge_index[1]
    px, py, pz = pos[:, 0], pos[:, 1], pos[:, 2]

    bf = lambda w: w.astype(jnp.bfloat16)
    ew = (bf(W_e1[:D]), bf(W_e1[D:2 * D]), bf(W_e1[2 * D + 1:]),
          W_e1[2 * D:2 * D + 1], b_e1.reshape(1, D),
          bf(W_e2), b_e2.reshape(1, D),
          bf(W_c1), b_c1.reshape(1, D), W_c2.reshape(1, D))

    outms, outcus = [], []
    for k in range(NSLICE):
        sl = slice(k * ESL, (k + 1) * ESL)
        row_k, col_k = row[sl], col[sl]
        idx2_k = jnp.concatenate([row_k, col_k])
        gathered, cdr = _sc_gather(h, idx2_k, px, py, pz, row_k, col_k)
        mvals, cus = _edge_mlp(gathered, edge_attr[sl], cdr, *ew)
        outm, outcu = _sc_scatter(mvals, cus, row_k)
        outms.append(outm)
        outcus.append(outcu)

    m_i = sum(o[0] + o[1] for o in outms)
    qacc = sum(o[0] + o[1] for o in outcus)
    q = qacc.reshape(CUN * 32, 4)[:N]
    h_new, pos_new = _node_mlp(m_i, q, h, pos,
                               bf(W_n1[:D]), bf(W_n1[D:]), b_n1.reshape(1, D),
                               bf(W_n2), b_n2.reshape(1, D))
    return (h_new, pos_new)


# speedup vs baseline: 1.7867x; 1.0305x over previous
import functools

import jax
import jax.numpy as jnp
from jax import lax
from jax.experimental import pallas as pl
from jax.experimental.pallas import tpu as pltpu
from jax.experimental.pallas import tpu_sc as plsc

N, E, D, DE = 10000, 320000, 128, 16
CUN = 320
NC, NS = 2, 16
NW = NC * NS
L = 16
CH = 128
ZCH = 80
ZCHUNKS = N // ZCH
CUCHUNKS = CUN // ZCH
NSLICE = 4
ESL = E // NSLICE


def _vector_mesh():
    return plsc.VectorSubcoreMesh(core_axis_name="c", subcore_axis_name="s")


_SC_PARAMS = pltpu.CompilerParams(needs_layout_passes=False)
_SC_PARAMS_UNTILED = pltpu.CompilerParams(needs_layout_passes=False,
                                          use_tc_tiling_on_sc=False)


GCH = 40
NB = 4


@jax.jit
def _sc_gather(table, idx2, px, py, pz, row, col):
    ne = row.shape[0]
    rpt = (2 * ne) // NW
    nch = rpt // GCH
    assert rpt % GCH == 0 and nch > 2 * NB
    echunks_all = ne // CH
    echunks = -(-echunks_all // NW)

    @functools.partial(
        pl.kernel,
        mesh=_vector_mesh(),
        compiler_params=_SC_PARAMS,
        out_type=[
            jax.ShapeDtypeStruct((2 * ne, D), jnp.float32),
            jax.ShapeDtypeStruct((8, ne), jnp.float32),
        ],
        scratch_types=[
            pltpu.VMEM((rpt,), jnp.int32),
        ] + [pltpu.VMEM((GCH, D), jnp.float32) for _ in range(NB)] + [
            pltpu.VMEM((N,), jnp.float32),
            pltpu.VMEM((N,), jnp.float32),
            pltpu.VMEM((N,), jnp.float32),
            pltpu.VMEM((CH,), jnp.int32),
            pltpu.VMEM((CH,), jnp.int32),
            pltpu.VMEM((8, CH), jnp.float32),
        ] + [pltpu.SemaphoreType.DMA for _ in range(2 * NB)],
    )
    def gk(table_hbm, idx_hbm, px_hbm, py_hbm, pz_hbm, row_hbm, col_hbm,
           out_hbm, cdr_hbm,
           idx_v, rb0, rb1, rb2, rb3, px_v, py_v, pz_v, r_v, c_v, geo_v,
           g0, g1, g2, g3, o0, o1, o2, o3):
        rbufs = (rb0, rb1, rb2, rb3)
        gsems = (g0, g1, g2, g3)
        osems = (o0, o1, o2, o3)
        wid = lax.axis_index("c") * NS + lax.axis_index("s")

        pltpu.sync_copy(px_hbm, px_v)
        pltpu.sync_copy(py_hbm, py_v)
        pltpu.sync_copy(pz_hbm, pz_v)

        zero16 = jnp.zeros((L,), jnp.float32)

        @pl.loop(5, 8)
        def _(r):
            @pl.loop(0, CH // L)
            def _(cc):
                geo_v[r, pl.ds(cc * L, L)] = zero16

        @pl.loop(0, echunks)
        def _(ch):
            cid = wid + ch * NW

            @pl.when(cid < echunks_all)
            def _():
                off = cid * CH
                pltpu.sync_copy(row_hbm.at[pl.ds(off, CH)], r_v)
                pltpu.sync_copy(col_hbm.at[pl.ds(off, CH)], c_v)

                @pl.loop(0, CH // L)
                def _(k):
                    sl = pl.ds(k * L, L)
                    ir = r_v[sl]
                    ic = c_v[sl]
                    dx = (plsc.load_gather(px_v, [ir])
                          - plsc.load_gather(px_v, [ic]))
                    dy = (plsc.load_gather(py_v, [ir])
                          - plsc.load_gather(py_v, [ic]))
                    dz = (plsc.load_gather(pz_v, [ir])
                          - plsc.load_gather(pz_v, [ic]))
                    geo_v[0, sl] = dx
                    geo_v[1, sl] = dy
                    geo_v[2, sl] = dz
                    geo_v[3, sl] = dx * dx + dy * dy + dz * dz
                    geo_v[4, sl] = lax.convert_element_type(
                        lax.bitwise_and(ir, 31), jnp.float32)

                pltpu.sync_copy(geo_v, cdr_hbm.at[:, pl.ds(off, CH)])

        gbase = wid * rpt
        pltpu.sync_copy(idx_hbm.at[pl.ds(gbase, rpt)], idx_v)

        def start_gather(cid, b):
            pltpu.async_copy(table_hbm.at[idx_v.at[pl.ds(cid * GCH, GCH)]],
                             rbufs[b], gsems[b])

        def start_out(cid, b):
            pltpu.async_copy(rbufs[b],
                             out_hbm.at[pl.ds(gbase + cid * GCH, GCH)],
                             osems[b])

        for b in range(NB):
            start_gather(b, b)

        @pl.loop(0, -(-nch // NB))
        def _(t):
            for b in range(NB):
                cid = t * NB + b

                @pl.when(cid < nch)
                def _():
                    pltpu.make_async_copy(table_hbm.at[pl.ds(0, GCH)],
                                          rbufs[b], gsems[b]).wait()
                    start_out(cid, b)

                @pl.when(cid + NB < nch)
                def _():
                    pltpu.make_async_copy(rbufs[b],
                                          out_hbm.at[pl.ds(0, GCH)],
                                          osems[b]).wait()
                    start_gather(cid + NB, b)

        for b in range(NB):
            pltpu.make_async_copy(rbufs[b], out_hbm.at[pl.ds(0, GCH)],
                                  osems[b]).wait()

    return gk(table, idx2, px, py, pz, row, col)


@jax.jit
def _sc_scatter(mvals, cus, row):
    ne = row.shape[0]
    echunks_all = ne // CH
    echunks = -(-echunks_all // NW)

    @functools.partial(
        pl.kernel,
        mesh=_vector_mesh(),
        compiler_params=_SC_PARAMS,
        out_type=[
            jax.ShapeDtypeStruct((NC, N, D), jnp.float32),
            jax.ShapeDtypeStruct((NC, CUN, D), jnp.float32),
        ],
        scratch_types=[
            pltpu.VMEM((CH,), jnp.int32),
            pltpu.VMEM((CH,), jnp.int32),
            pltpu.VMEM((CH,), jnp.int32),
            pltpu.VMEM((CH, D), jnp.float32),
            pltpu.VMEM((CH, D), jnp.float32),
            pltpu.VMEM((8, CH), jnp.float32),
            pltpu.VMEM((8, CH), jnp.float32),
            pltpu.VMEM_SHARED((N, D), jnp.float32),
            pltpu.VMEM_SHARED((CUN, D), jnp.float32),
            pltpu.SemaphoreType.DMA,
            pltpu.SemaphoreType.DMA,
        ],
    )
    def sk(mvals_hbm, cus_hbm, idx_hbm, outm_hbm, outcu_hbm,
           cuidx_v, idx_v0, idx_v1, mv, cuv, cus_v0, cus_v1,
           macc, cuacc, ls0, ls1):
        c = lax.axis_index("c")
        s = lax.axis_index("s")
        wid = c * NS + s
        idx_vs = (idx_v0, idx_v1)
        mvs = (mv, mv)
        cus_vs = (cus_v0, cus_v1)
        lsems = (ls0, ls1)

        zero16 = jnp.zeros((L,), jnp.float32)
        one16 = jnp.ones((L,), jnp.float32)

        @pl.loop(0, CH)
        def _(r):
            @pl.loop(0, D // L)
            def _(cc):
                mv[r, pl.ds(cc * L, L)] = zero16
                cuv[r, pl.ds(cc * L, L)] = zero16

        @pl.loop(0, ZCHUNKS)
        def _(z):
            @pl.when(lax.rem(z, NS) == s)
            def _():
                pltpu.sync_copy(mv.at[pl.ds(0, ZCH)],
                                macc.at[pl.ds(z * ZCH, ZCH)])

        @pl.loop(0, CUCHUNKS)
        def _(z):
            @pl.when(z == s)
            def _():
                pltpu.sync_copy(mv.at[pl.ds(0, ZCH)],
                                cuacc.at[pl.ds(z * ZCH, ZCH)])

        plsc.subcore_barrier()

        lanes16 = lax.iota(jnp.int32, L)

        def start_loads(cid, b):
            off = cid * CH
            pltpu.async_copy(idx_hbm.at[pl.ds(off, CH)], idx_vs[b], lsems[b])
            pltpu.async_copy(cus_hbm.at[:, pl.ds(off, CH)], cus_vs[b],
                             lsems[b])

        def wait_loads(b):
            pltpu.make_async_copy(idx_hbm.at[pl.ds(0, CH)], idx_vs[b],
                                  lsems[b]).wait()
            pltpu.make_async_copy(cus_hbm.at[:, pl.ds(0, CH)], cus_vs[b],
                                  lsems[b]).wait()

        for b in range(2):
            @pl.when(wid + b * NW < echunks_all)
            def _():
                start_loads(wid + b * NW, b)

        @pl.loop(0, -(-echunks // 2))
        def _(t):
            for b in range(2):
                ch = 2 * t + b
                cid = wid + ch * NW

                @pl.when((ch < echunks) & (cid < echunks_all))
                def _():
                    pltpu.sync_copy(mvals_hbm.at[pl.ds(cid * CH, CH)], mv)
                    wait_loads(b)

                    @pl.loop(0, CH // L)
                    def _(k):
                        sl = pl.ds(k * L, L)
                        r16 = idx_vs[b][sl]
                        rowi = lanes16 + k * L
                        lane = lax.shift_left(lax.bitwise_and(r16, 31), 2)
                        plsc.store_scatter(cuv, [rowi, lane], cus_vs[b][0, sl])
                        plsc.store_scatter(cuv, [rowi, lane + 1],
                                           cus_vs[b][1, sl])
                        plsc.store_scatter(cuv, [rowi, lane + 2],
                                           cus_vs[b][2, sl])
                        plsc.store_scatter(cuv, [rowi, lane + 3], one16)
                        cuidx_v[sl] = lax.shift_right_logical(r16, 5)

                    pltpu.sync_copy(mvs[b], macc.at[idx_vs[b]], add=True)
                    pltpu.sync_copy(cuv, cuacc.at[cuidx_v], add=True)

                    @pl.loop(0, CH // L)
                    def _(k):
                        sl = pl.ds(k * L, L)
                        r16 = idx_vs[b][sl]
                        rowi = lanes16 + k * L
                        lane = lax.shift_left(lax.bitwise_and(r16, 31), 2)
                        plsc.store_scatter(cuv, [rowi, lane], zero16)
                        plsc.store_scatter(cuv, [rowi, lane + 1], zero16)
                        plsc.store_scatter(cuv, [rowi, lane + 2], zero16)
                        plsc.store_scatter(cuv, [rowi, lane + 3], zero16)

                    cid2 = cid + 2 * NW

                    @pl.when((ch + 2 < echunks) & (cid2 < echunks_all))
                    def _():
                        start_loads(cid2, b)

        plsc.subcore_barrier()

        @pl.loop(0, ZCHUNKS)
        def _(z):
            @pl.when(lax.rem(z, NS) == s)
            def _():
                pltpu.sync_copy(macc.at[pl.ds(z * ZCH, ZCH)],
                                outm_hbm.at[c, pl.ds(z * ZCH, ZCH)])

        @pl.loop(0, CUCHUNKS)
        def _(z):
            @pl.when(z == s)
            def _():
                pltpu.sync_copy(cuacc.at[pl.ds(z * ZCH, ZCH)],
                                outcu_hbm.at[c, pl.ds(z * ZCH, ZCH)])

    return sk(mvals, cus, row)


BE = 3200


def _silu(x):
    return x * jax.nn.sigmoid(x)


def _bf(x):
    return x.astype(jnp.bfloat16)


def _edge_body(gr_ref, gc_ref, ea_ref, cdr_ref, w1r_ref, w1c_ref, w1a_ref,
               w1rad_ref, b1_ref, w2_ref, b2_ref, wc1_ref, bc1_ref, wc2_ref,
               m_ref, cus_ref):
    eye8 = jnp.eye(8, dtype=jnp.float32)
    cdrT = lax.dot_general(cdr_ref[...], eye8, (((0,), (0,)), ((), ())),
                           preferred_element_type=jnp.float32)
    cd = cdrT[:, 0:3]
    radial = cdrT[:, 3:4]

    t1 = (jnp.dot(_bf(gr_ref[...]), w1r_ref[...],
                  preferred_element_type=jnp.float32)
          + jnp.dot(_bf(gc_ref[...]), w1c_ref[...],
                    preferred_element_type=jnp.float32)
          + jnp.dot(_bf(ea_ref[...]), w1a_ref[...],
                    preferred_element_type=jnp.float32)
          + radial * w1rad_ref[...]
          + b1_ref[...])
    x = _silu(t1)
    m = _silu(jnp.dot(_bf(x), w2_ref[...], preferred_element_type=jnp.float32)
              + b2_ref[...])
    c1 = _silu(jnp.dot(_bf(m), wc1_ref[...], preferred_element_type=jnp.float32)
               + bc1_ref[...])
    w = jnp.sum(c1 * wc2_ref[...], axis=1, keepdims=True)
    cu = cd * (w * lax.rsqrt(radial + 1e-8))

    m_ref[...] = m
    cuT = jnp.concatenate([cu, jnp.zeros((BE, 5), jnp.float32)], axis=1)
    cus_ref[...] = lax.dot_general(eye8, cuT, (((1,), (1,)), ((), ())),
                                   preferred_element_type=jnp.float32)


def _edge_mlp(gathered, edge_attr, cdr, w1r, w1c, w1a, w1rad, b1, w2, b2,
              wc1, bc1, wc2):
    ne = edge_attr.shape[0]
    nb = ne // BE
    full = lambda shape: pl.BlockSpec(shape, lambda i: (0, 0))
    return pl.pallas_call(
        _edge_body,
        grid=(nb,),
        in_specs=[
            pl.BlockSpec((BE, D), lambda i: (i, 0)),
            pl.BlockSpec((BE, D), lambda i: (i + nb, 0)),
            pl.BlockSpec((BE, DE), lambda i: (i, 0)),
            pl.BlockSpec((8, BE), lambda i: (0, i)),
            full((D, D)), full((D, D)), full((DE, D)), full((1, D)),
            full((1, D)), full((D, D)), full((1, D)),
            full((D, D)), full((1, D)), full((1, D)),
        ],
        out_specs=[
            pl.BlockSpec((BE, D), lambda i: (i, 0)),
            pl.BlockSpec((8, BE), lambda i: (0, i)),
        ],
        out_shape=[
            jax.ShapeDtypeStruct((ne, D), jnp.float32),
            jax.ShapeDtypeStruct((8, ne), jnp.float32),
        ],
    )(gathered, gathered, edge_attr, cdr, w1r, w1c, w1a, w1rad, b1, w2, b2,
      wc1, bc1, wc2)


BN = 2000


def _node_body(p_ref, q_ref, h_ref, pos_ref, wn1h_ref, wn1m_ref, bn1_ref,
               wn2_ref, bn2_ref, hnew_ref, posnew_ref):
    m_i = p_ref[...]
    q = q_ref[...]
    pu = q[:, 0:3]
    cnt = q[:, 3:4]
    h = h_ref[...]
    t = _silu(jnp.dot(_bf(h), wn1h_ref[...], preferred_element_type=jnp.float32)
              + jnp.dot(_bf(m_i), wn1m_ref[...],
                        preferred_element_type=jnp.float32)
              + bn1_ref[...])
    hnew_ref[...] = h + jnp.dot(_bf(t), wn2_ref[...],
                                preferred_element_type=jnp.float32) + bn2_ref[...]
    posnew_ref[...] = pos_ref[...] + pu / (cnt + 1e-6)


def _node_mlp(p, q, h, pos, wn1h, wn1m, bn1, wn2, bn2):
    nb = N // BN
    full = lambda shape: pl.BlockSpec(shape, lambda i: (0, 0))
    return pl.pallas_call(
        _node_body,
        grid=(nb,),
        in_specs=[
            pl.BlockSpec((BN, D), lambda i: (i, 0)),
            pl.BlockSpec((BN, 4), lambda i: (i, 0)),
            pl.BlockSpec((BN, D), lambda i: (i, 0)),
            pl.BlockSpec((BN, 3), lambda i: (i, 0)),
            full((D, D)), full((D, D)), full((1, D)),
            full((D, D)), full((1, D)),
        ],
        out_specs=[
            pl.BlockSpec((BN, D), lambda i: (i, 0)),
            pl.BlockSpec((BN, 3), lambda i: (i, 0)),
        ],
        out_shape=[
            jax.ShapeDtypeStruct((N, D), jnp.float32),
            jax.ShapeDtypeStruct((N, 3), jnp.float32),
        ],
    )(p, q, h, pos, wn1h, wn1m, bn1, wn2, bn2)


def kernel(h, pos, edge_index, edge_attr, W_e1, b_e1, W_e2, b_e2,
           W_n1, b_n1, W_n2, b_n2, W_c1, b_c1, W_c2):
    row, col = edge_index[0], edge_index[1]
    px, py, pz = pos[:, 0], pos[:, 1], pos[:, 2]

    bf = lambda w: w.astype(jnp.bfloat16)
    ew = (bf(W_e1[:D]), bf(W_e1[D:2 * D]), bf(W_e1[2 * D + 1:]),
          W_e1[2 * D:2 * D + 1], b_e1.reshape(1, D),
          bf(W_e2), b_e2.reshape(1, D),
          bf(W_c1), b_c1.reshape(1, D), W_c2.reshape(1, D))

    outms, outcus = [], []
    for k in range(NSLICE):
        sl = slice(k * ESL, (k + 1) * ESL)
        row_k, col_k = row[sl], col[sl]
        idx2_k = jnp.concatenate([row_k, col_k])
        gathered, cdr = _sc_gather(h, idx2_k, px, py, pz, row_k, col_k)
        mvals, cus = _edge_mlp(gathered, edge_attr[sl], cdr, *ew)
        outm, outcu = _sc_scatter(mvals, cus, row_k)
        outms.append(outm)
        outcus.append(outcu)

    m_i = sum(o[0] + o[1] for o in outms)
    qacc = sum(o[0] + o[1] for o in outcus)
    q = qacc.reshape(CUN * 32, 4)[:N]
    h_new, pos_new = _node_mlp(m_i, q, h, pos,
                               bf(W_n1[:D]), bf(W_n1[D:]), b_n1.reshape(1, D),
                               bf(W_n2), b_n2.reshape(1, D))
    return (h_new, pos_new)


# speedup vs baseline: 1.8011x; 1.0081x over previous
import functools

import jax
import jax.numpy as jnp
from jax import lax
from jax.experimental import pallas as pl
from jax.experimental.pallas import tpu as pltpu
from jax.experimental.pallas import tpu_sc as plsc

N, E, D, DE = 10000, 320000, 128, 16
CUN = 320
NC, NS = 2, 16
NW = NC * NS
L = 16
CH = 128
ZCH = 80
ZCHUNKS = N // ZCH
CUCHUNKS = CUN // ZCH
NSLICE = 4
ESL = E // NSLICE


def _vector_mesh():
    return plsc.VectorSubcoreMesh(core_axis_name="c", subcore_axis_name="s")


_SC_PARAMS = pltpu.CompilerParams(needs_layout_passes=False)
_SC_PARAMS_UNTILED = pltpu.CompilerParams(needs_layout_passes=False,
                                          use_tc_tiling_on_sc=False)


GCH = 40
NB = 4


@jax.jit
def _sc_gather(table, idx2, px, py, pz, row, col):
    ne = row.shape[0]
    rpt = (2 * ne) // NW
    nch = rpt // GCH
    assert rpt % GCH == 0 and nch > 2 * NB
    echunks_all = ne // CH
    echunks = -(-echunks_all // NW)

    @functools.partial(
        pl.kernel,
        mesh=_vector_mesh(),
        compiler_params=_SC_PARAMS,
        out_type=[
            jax.ShapeDtypeStruct((2 * ne, D), jnp.float32),
            jax.ShapeDtypeStruct((8, ne), jnp.float32),
        ],
        scratch_types=[
            pltpu.VMEM((rpt,), jnp.int32),
        ] + [pltpu.VMEM((GCH, D), jnp.float32) for _ in range(NB)] + [
            pltpu.VMEM((N,), jnp.float32),
            pltpu.VMEM((N,), jnp.float32),
            pltpu.VMEM((N,), jnp.float32),
            pltpu.VMEM((CH,), jnp.int32),
            pltpu.VMEM((CH,), jnp.int32),
            pltpu.VMEM((CH,), jnp.int32),
            pltpu.VMEM((CH,), jnp.int32),
            pltpu.VMEM((8, CH), jnp.float32),
            pltpu.VMEM((8, CH), jnp.float32),
        ] + [pltpu.SemaphoreType.DMA for _ in range(2 * NB + 4)],
    )
    def gk(table_hbm, idx_hbm, px_hbm, py_hbm, pz_hbm, row_hbm, col_hbm,
           out_hbm, cdr_hbm,
           idx_v, rb0, rb1, rb2, rb3, px_v, py_v, pz_v, r_v0, r_v1,
           c_v0, c_v1, geo_v0, geo_v1,
           g0, g1, g2, g3, o0, o1, o2, o3, e0, e1, l0, l1):
        rbufs = (rb0, rb1, rb2, rb3)
        gsems = (g0, g1, g2, g3)
        osems = (o0, o1, o2, o3)
        r_vs = (r_v0, r_v1)
        c_vs = (c_v0, c_v1)
        geo_vs = (geo_v0, geo_v1)
        esems = (e0, e1)
        lsems = (l0, l1)
        wid = lax.axis_index("c") * NS + lax.axis_index("s")

        pltpu.sync_copy(px_hbm, px_v)
        pltpu.sync_copy(py_hbm, py_v)
        pltpu.sync_copy(pz_hbm, pz_v)

        zero16 = jnp.zeros((L,), jnp.float32)

        for gv in geo_vs:
            @pl.loop(5, 8)
            def _(r):
                @pl.loop(0, CH // L)
                def _(cc):
                    gv[r, pl.ds(cc * L, L)] = zero16

        def start_eloads(cid, b):
            off = cid * CH
            pltpu.async_copy(row_hbm.at[pl.ds(off, CH)], r_vs[b], lsems[b])
            pltpu.async_copy(col_hbm.at[pl.ds(off, CH)], c_vs[b], lsems[b])

        def wait_eloads(b):
            pltpu.make_async_copy(row_hbm.at[pl.ds(0, CH)], r_vs[b],
                                  lsems[b]).wait()
            pltpu.make_async_copy(col_hbm.at[pl.ds(0, CH)], c_vs[b],
                                  lsems[b]).wait()

        for b in range(2):
            @pl.when(wid + b * NW < echunks_all)
            def _():
                start_eloads(wid + b * NW, b)

        @pl.loop(0, -(-echunks // 2))
        def _(t):
            for b in range(2):
                ch = 2 * t + b
                cid = wid + ch * NW

                @pl.when(cid < echunks_all)
                def _():
                    wait_eloads(b)

                    @pl.when(ch >= 2)
                    def _():
                        pltpu.make_async_copy(
                            geo_vs[b], cdr_hbm.at[:, pl.ds(0, CH)],
                            esems[b]).wait()

                    @pl.loop(0, CH // L)
                    def _(k):
                        sl = pl.ds(k * L, L)
                        ir = r_vs[b][sl]
                        ic = c_vs[b][sl]
                        dx = (plsc.load_gather(px_v, [ir])
                              - plsc.load_gather(px_v, [ic]))
                        dy = (plsc.load_gather(py_v, [ir])
                              - plsc.load_gather(py_v, [ic]))
                        dz = (plsc.load_gather(pz_v, [ir])
                              - plsc.load_gather(pz_v, [ic]))
                        geo_vs[b][0, sl] = dx
                        geo_vs[b][1, sl] = dy
                        geo_vs[b][2, sl] = dz
                        geo_vs[b][3, sl] = dx * dx + dy * dy + dz * dz
                        geo_vs[b][4, sl] = lax.convert_element_type(
                            lax.bitwise_and(ir, 31), jnp.float32)

                    pltpu.async_copy(geo_vs[b],
                                     cdr_hbm.at[:, pl.ds(cid * CH, CH)],
                                     esems[b])

                    @pl.when(cid + 2 * NW < echunks_all)
                    def _():
                        start_eloads(cid + 2 * NW, b)

        for b in range(2):
            @pl.when(wid + b * NW < echunks_all)
            def _():
                pltpu.make_async_copy(geo_vs[b], cdr_hbm.at[:, pl.ds(0, CH)],
                                      esems[b]).wait()

        gbase = wid * rpt
        pltpu.sync_copy(idx_hbm.at[pl.ds(gbase, rpt)], idx_v)

        def start_gather(cid, b):
            pltpu.async_copy(table_hbm.at[idx_v.at[pl.ds(cid * GCH, GCH)]],
                             rbufs[b], gsems[b])

        def start_out(cid, b):
            pltpu.async_copy(rbufs[b],
                             out_hbm.at[pl.ds(gbase + cid * GCH, GCH)],
                             osems[b])

        for b in range(NB):
            start_gather(b, b)

        @pl.loop(0, -(-nch // NB))
        def _(t):
            for b in range(NB):
                cid = t * NB + b

                @pl.when(cid < nch)
                def _():
                    pltpu.make_async_copy(table_hbm.at[pl.ds(0, GCH)],
                                          rbufs[b], gsems[b]).wait()
                    start_out(cid, b)

                @pl.when(cid + NB < nch)
                def _():
                    pltpu.make_async_copy(rbufs[b],
                                          out_hbm.at[pl.ds(0, GCH)],
                                          osems[b]).wait()
                    start_gather(cid + NB, b)

        for b in range(NB):
            pltpu.make_async_copy(rbufs[b], out_hbm.at[pl.ds(0, GCH)],
                                  osems[b]).wait()

    return gk(table, idx2, px, py, pz, row, col)


@jax.jit
def _sc_scatter(mvals, cus, row):
    ne = row.shape[0]
    echunks_all = ne // CH
    echunks = -(-echunks_all // NW)

    @functools.partial(
        pl.kernel,
        mesh=_vector_mesh(),
        compiler_params=_SC_PARAMS,
        out_type=[
            jax.ShapeDtypeStruct((NC, N, D), jnp.float32),
            jax.ShapeDtypeStruct((NC, CUN, D), jnp.float32),
        ],
        scratch_types=[
            pltpu.VMEM((CH,), jnp.int32),
            pltpu.VMEM((CH,), jnp.int32),
            pltpu.VMEM((CH,), jnp.int32),
            pltpu.VMEM((CH, D), jnp.float32),
            pltpu.VMEM((CH, D), jnp.float32),
            pltpu.VMEM((8, CH), jnp.float32),
            pltpu.VMEM((8, CH), jnp.float32),
            pltpu.VMEM_SHARED((N, D), jnp.float32),
            pltpu.VMEM_SHARED((CUN, D), jnp.float32),
            pltpu.SemaphoreType.DMA,
            pltpu.SemaphoreType.DMA,
        ],
    )
    def sk(mvals_hbm, cus_hbm, idx_hbm, outm_hbm, outcu_hbm,
           cuidx_v, idx_v0, idx_v1, mv, cuv, cus_v0, cus_v1,
           macc, cuacc, ls0, ls1):
        c = lax.axis_index("c")
        s = lax.axis_index("s")
        wid = c * NS + s
        idx_vs = (idx_v0, idx_v1)
        mvs = (mv, mv)
        cus_vs = (cus_v0, cus_v1)
        lsems = (ls0, ls1)

        zero16 = jnp.zeros((L,), jnp.float32)
        one16 = jnp.ones((L,), jnp.float32)

        @pl.loop(0, CH)
        def _(r):
            @pl.loop(0, D // L)
            def _(cc):
                mv[r, pl.ds(cc * L, L)] = zero16
                cuv[r, pl.ds(cc * L, L)] = zero16

        @pl.loop(0, ZCHUNKS)
        def _(z):
            @pl.when(lax.rem(z, NS) == s)
            def _():
                pltpu.sync_copy(mv.at[pl.ds(0, ZCH)],
                                macc.at[pl.ds(z * ZCH, ZCH)])

        @pl.loop(0, CUCHUNKS)
        def _(z):
            @pl.when(z == s)
            def _():
                pltpu.sync_copy(mv.at[pl.ds(0, ZCH)],
                                cuacc.at[pl.ds(z * ZCH, ZCH)])

        plsc.subcore_barrier()

        lanes16 = lax.iota(jnp.int32, L)

        def start_loads(cid, b):
            off = cid * CH
            pltpu.async_copy(idx_hbm.at[pl.ds(off, CH)], idx_vs[b], lsems[b])
            pltpu.async_copy(cus_hbm.at[:, pl.ds(off, CH)], cus_vs[b],
                             lsems[b])

        def wait_loads(b):
            pltpu.make_async_copy(idx_hbm.at[pl.ds(0, CH)], idx_vs[b],
                                  lsems[b]).wait()
            pltpu.make_async_copy(cus_hbm.at[:, pl.ds(0, CH)], cus_vs[b],
                                  lsems[b]).wait()

        for b in range(2):
            @pl.when(wid + b * NW < echunks_all)
            def _():
                start_loads(wid + b * NW, b)

        @pl.loop(0, -(-echunks // 2))
        def _(t):
            for b in range(2):
                ch = 2 * t + b
                cid = wid + ch * NW

                @pl.when((ch < echunks) & (cid < echunks_all))
                def _():
                    pltpu.sync_copy(mvals_hbm.at[pl.ds(cid * CH, CH)], mv)
                    wait_loads(b)

                    @pl.loop(0, CH // L)
                    def _(k):
                        sl = pl.ds(k * L, L)
                        r16 = idx_vs[b][sl]
                        rowi = lanes16 + k * L
                        lane = lax.shift_left(lax.bitwise_and(r16, 31), 2)
                        plsc.store_scatter(cuv, [rowi, lane], cus_vs[b][0, sl])
                        plsc.store_scatter(cuv, [rowi, lane + 1],
                                           cus_vs[b][1, sl])
                        plsc.store_scatter(cuv, [rowi, lane + 2],
                                           cus_vs[b][2, sl])
                        plsc.store_scatter(cuv, [rowi, lane + 3], one16)
                        cuidx_v[sl] = lax.shift_right_logical(r16, 5)

                    pltpu.sync_copy(mvs[b], macc.at[idx_vs[b]], add=True)
                    pltpu.sync_copy(cuv, cuacc.at[cuidx_v], add=True)

                    @pl.loop(0, CH // L)
                    def _(k):
                        sl = pl.ds(k * L, L)
                        r16 = idx_vs[b][sl]
                        rowi = lanes16 + k * L
                        lane = lax.shift_left(lax.bitwise_and(r16, 31), 2)
                        plsc.store_scatter(cuv, [rowi, lane], zero16)
                        plsc.store_scatter(cuv, [rowi, lane + 1], zero16)
                        plsc.store_scatter(cuv, [rowi, lane + 2], zero16)
                        plsc.store_scatter(cuv, [rowi, lane + 3], zero16)

                    cid2 = cid + 2 * NW

                    @pl.when((ch + 2 < echunks) & (cid2 < echunks_all))
                    def _():
                        start_loads(cid2, b)

        plsc.subcore_barrier()

        @pl.loop(0, ZCHUNKS)
        def _(z):
            @pl.when(lax.rem(z, NS) == s)
            def _():
                pltpu.sync_copy(macc.at[pl.ds(z * ZCH, ZCH)],
                                outm_hbm.at[c, pl.ds(z * ZCH, ZCH)])

        @pl.loop(0, CUCHUNKS)
        def _(z):
            @pl.when(z == s)
            def _():
                pltpu.sync_copy(cuacc.at[pl.ds(z * ZCH, ZCH)],
                                outcu_hbm.at[c, pl.ds(z * ZCH, ZCH)])

    return sk(mvals, cus, row)


BE = 3200


def _silu(x):
    return x * jax.nn.sigmoid(x)


def _bf(x):
    return x.astype(jnp.bfloat16)


def _edge_body(gr_ref, gc_ref, ea_ref, cdr_ref, w1r_ref, w1c_ref, w1a_ref,
               w1rad_ref, b1_ref, w2_ref, b2_ref, wc1_ref, bc1_ref, wc2_ref,
               m_ref, cus_ref):
    eye8 = jnp.eye(8, dtype=jnp.float32)
    cdrT = lax.dot_general(cdr_ref[...], eye8, (((0,), (0,)), ((), ())),
                           preferred_element_type=jnp.float32)
    cd = cdrT[:, 0:3]
    radial = cdrT[:, 3:4]

    t1 = (jnp.dot(_bf(gr_ref[...]), w1r_ref[...],
                  preferred_element_type=jnp.float32)
          + jnp.dot(_bf(gc_ref[...]), w1c_ref[...],
                    preferred_element_type=jnp.float32)
          + jnp.dot(_bf(ea_ref[...]), w1a_ref[...],
                    preferred_element_type=jnp.float32)
          + radial * w1rad_ref[...]
          + b1_ref[...])
    x = _silu(t1)
    m = _silu(jnp.dot(_bf(x), w2_ref[...], preferred_element_type=jnp.float32)
              + b2_ref[...])
    c1 = _silu(jnp.dot(_bf(m), wc1_ref[...], preferred_element_type=jnp.float32)
               + bc1_ref[...])
    w = jnp.sum(c1 * wc2_ref[...], axis=1, keepdims=True)
    cu = cd * (w * lax.rsqrt(radial + 1e-8))

    m_ref[...] = m
    cuT = jnp.concatenate([cu, jnp.zeros((BE, 5), jnp.float32)], axis=1)
    cus_ref[...] = lax.dot_general(eye8, cuT, (((1,), (1,)), ((), ())),
                                   preferred_element_type=jnp.float32)


def _edge_mlp(gathered, edge_attr, cdr, w1r, w1c, w1a, w1rad, b1, w2, b2,
              wc1, bc1, wc2):
    ne = edge_attr.shape[0]
    nb = ne // BE
    full = lambda shape: pl.BlockSpec(shape, lambda i: (0, 0))
    return pl.pallas_call(
        _edge_body,
        grid=(nb,),
        in_specs=[
            pl.BlockSpec((BE, D), lambda i: (i, 0)),
            pl.BlockSpec((BE, D), lambda i: (i + nb, 0)),
            pl.BlockSpec((BE, DE), lambda i: (i, 0)),
            pl.BlockSpec((8, BE), lambda i: (0, i)),
            full((D, D)), full((D, D)), full((DE, D)), full((1, D)),
            full((1, D)), full((D, D)), full((1, D)),
            full((D, D)), full((1, D)), full((1, D)),
        ],
        out_specs=[
            pl.BlockSpec((BE, D), lambda i: (i, 0)),
            pl.BlockSpec((8, BE), lambda i: (0, i)),
        ],
        out_shape=[
            jax.ShapeDtypeStruct((ne, D), jnp.float32),
            jax.ShapeDtypeStruct((8, ne), jnp.float32),
        ],
    )(gathered, gathered, edge_attr, cdr, w1r, w1c, w1a, w1rad, b1, w2, b2,
      wc1, bc1, wc2)


BN = 2000


def _node_body(p_ref, q_ref, h_ref, pos_ref, wn1h_ref, wn1m_ref, bn1_ref,
               wn2_ref, bn2_ref, hnew_ref, posnew_ref):
    m_i = p_ref[...]
    q = q_ref[...]
    pu = q[:, 0:3]
    cnt = q[:, 3:4]
    h = h_ref[...]
    t = _silu(jnp.dot(_bf(h), wn1h_ref[...], preferred_element_type=jnp.float32)
              + jnp.dot(_bf(m_i), wn1m_ref[...],
                        preferred_element_type=jnp.float32)
              + bn1_ref[...])
    hnew_ref[...] = h + jnp.dot(_bf(t), wn2_ref[...],
                                preferred_element_type=jnp.float32) + bn2_ref[...]
    posnew_ref[...] = pos_ref[...] + pu / (cnt + 1e-6)


def _node_mlp(p, q, h, pos, wn1h, wn1m, bn1, wn2, bn2):
    nb = N // BN
    full = lambda shape: pl.BlockSpec(shape, lambda i: (0, 0))
    return pl.pallas_call(
        _node_body,
        grid=(nb,),
        in_specs=[
            pl.BlockSpec((BN, D), lambda i: (i, 0)),
            pl.BlockSpec((BN, 4), lambda i: (i, 0)),
            pl.BlockSpec((BN, D), lambda i: (i, 0)),
            pl.BlockSpec((BN, 3), lambda i: (i, 0)),
            full((D, D)), full((D, D)), full((1, D)),
            full((D, D)), full((1, D)),
        ],
        out_specs=[
            pl.BlockSpec((BN, D), lambda i: (i, 0)),
            pl.BlockSpec((BN, 3), lambda i: (i, 0)),
        ],
        out_shape=[
            jax.ShapeDtypeStruct((N, D), jnp.float32),
            jax.ShapeDtypeStruct((N, 3), jnp.float32),
        ],
    )(p, q, h, pos, wn1h, wn1m, bn1, wn2, bn2)


def kernel(h, pos, edge_index, edge_attr, W_e1, b_e1, W_e2, b_e2,
           W_n1, b_n1, W_n2, b_n2, W_c1, b_c1, W_c2):
    row, col = edge_index[0], edge_index[1]
    px, py, pz = pos[:, 0], pos[:, 1], pos[:, 2]

    bf = lambda w: w.astype(jnp.bfloat16)
    ew = (bf(W_e1[:D]), bf(W_e1[D:2 * D]), bf(W_e1[2 * D + 1:]),
          W_e1[2 * D:2 * D + 1], b_e1.reshape(1, D),
          bf(W_e2), b_e2.reshape(1, D),
          bf(W_c1), b_c1.reshape(1, D), W_c2.reshape(1, D))

    outms, outcus = [], []
    for k in range(NSLICE):
        sl = slice(k * ESL, (k + 1) * ESL)
        row_k, col_k = row[sl], col[sl]
        idx2_k = jnp.concatenate([row_k, col_k])
        gathered, cdr = _sc_gather(h, idx2_k, px, py, pz, row_k, col_k)
        mvals, cus = _edge_mlp(gathered, edge_attr[sl], cdr, *ew)
        outm, outcu = _sc_scatter(mvals, cus, row_k)
        outms.append(outm)
        outcus.append(outcu)

    m_i = sum(o[0] + o[1] for o in outms)
    qacc = sum(o[0] + o[1] for o in outcus)
    q = qacc.reshape(CUN * 32, 4)[:N]
    h_new, pos_new = _node_mlp(m_i, q, h, pos,
                               bf(W_n1[:D]), bf(W_n1[D:]), b_n1.reshape(1, D),
                               bf(W_n2), b_n2.reshape(1, D))
    return (h_new, pos_new)
